# baseline (speedup 1.0000x reference)
import functools

import jax
import jax.numpy as jnp
from jax import lax
from jax.experimental import pallas as pl
from jax.experimental.pallas import tpu as pltpu
from jax.experimental.pallas import tpu_sc as plsc

_VOCAB = 100000
_EMBED = 300
_N_TOKENS = 16384

_NC = 2
_NS = 16
_NW = _NC * _NS
_TPW = _N_TOKENS // _NW
_CH = 128
_NCHUNK = _TPW // _CH


def _sc_gather(idx, table):
    mesh = plsc.VectorSubcoreMesh(core_axis_name="c", subcore_axis_name="s")

    @functools.partial(
        pl.kernel,
        mesh=mesh,
        out_type=jax.ShapeDtypeStruct((_N_TOKENS, _EMBED), jnp.float32),
        scratch_types=[
            pltpu.VMEM((_CH,), jnp.int32),
            pltpu.VMEM((_CH, _EMBED), jnp.float32),
            pltpu.SemaphoreType.DMA,
        ],
        compiler_params=pltpu.CompilerParams(use_tc_tiling_on_sc=False),
    )
    def k(idx_hbm, table_hbm, out_hbm, idx_v, rows_v, sem):
        wid = lax.axis_index("s") * _NC + lax.axis_index("c")
        base = wid * _TPW
        for j in range(_NCHUNK):
            off = base + j * _CH
            pltpu.sync_copy(idx_hbm.at[pl.ds(off, _CH)], idx_v)
            pltpu.async_copy(table_hbm.at[idx_v], rows_v, sem).wait()
            pltpu.sync_copy(rows_v, out_hbm.at[pl.ds(off, _CH)])

    return k(idx, table)


_TB = 1024


def _tc_transpose(x):
    def body(x_ref, o_ref):
        o_ref[...] = x_ref[...].T

    return pl.pallas_call(
        body,
        grid=(_N_TOKENS // _TB,),
        in_specs=[pl.BlockSpec((_TB, _EMBED), lambda i: (i, 0))],
        out_specs=pl.BlockSpec((_EMBED, _TB), lambda i: (0, i)),
        out_shape=jax.ShapeDtypeStruct((_EMBED, _N_TOKENS), jnp.float32),
    )(x)


def kernel(input, table):
    idx = input.astype(jnp.int32)
    gathered = _sc_gather(idx, table)
    return gathered.T


# speedup vs baseline: 3.2940x; 3.2940x over previous
import functools

import jax
import jax.numpy as jnp
from jax import lax
from jax.experimental import pallas as pl
from jax.experimental.pallas import tpu as pltpu
from jax.experimental.pallas import tpu_sc as plsc

_VOCAB = 100000
_EMBED = 300
_N_TOKENS = 16384
_EPAD = 384

_NC = 2
_NS = 16
_NW = _NC * _NS
_TPW = _N_TOKENS // _NW
_CH = 128
_NCHUNK = _TPW // _CH


def _sc_gather(idx, table, tail):
    mesh = plsc.VectorSubcoreMesh(core_axis_name="c", subcore_axis_name="s")

    @functools.partial(
        pl.kernel,
        mesh=mesh,
        out_type=jax.ShapeDtypeStruct((_N_TOKENS, _EPAD), jnp.float32),
        scratch_types=[
            pltpu.VMEM((_CH,), jnp.int32),
            pltpu.VMEM((_CH, 128), jnp.float32),
            pltpu.SemaphoreType.DMA,
        ],
        compiler_params=pltpu.CompilerParams(use_tc_tiling_on_sc=True),
    )
    def k(idx_hbm, table_hbm, tail_hbm, out_hbm, idx_v, rows_v, sem):
        wid = lax.axis_index("s") * _NC + lax.axis_index("c")
        base = wid * _TPW
        for j in range(_NCHUNK):
            off = base + j * _CH
            pltpu.sync_copy(idx_hbm.at[pl.ds(off, _CH)], idx_v)
            for p in range(3):
                if p < 2:
                    src = table_hbm.at[idx_v, pl.ds(p * 128, 128)]
                else:
                    src = tail_hbm.at[idx_v]
                pltpu.async_copy(src, rows_v, sem).wait()
                pltpu.sync_copy(
                    rows_v, out_hbm.at[pl.ds(off, _CH), pl.ds(p * 128, 128)])

    return k(idx, table, tail)


_TB = 1024


def _tc_transpose(x):
    def body(x_ref, o_ref):
        o_ref[...] = x_ref[:, :_EMBED].T

    return pl.pallas_call(
        body,
        grid=(_N_TOKENS // _TB,),
        in_specs=[pl.BlockSpec((_TB, _EPAD), lambda i: (i, 0))],
        out_specs=pl.BlockSpec((_EMBED, _TB), lambda i: (0, i)),
        out_shape=jax.ShapeDtypeStruct((_EMBED, _N_TOKENS), jnp.float32),
    )(x)


def kernel(input, table):
    idx = input.astype(jnp.int32)
    tail = jnp.pad(lax.slice_in_dim(table, 2 * 128, _EMBED, axis=1),
                   ((0, 0), (0, _EPAD - _EMBED)))
    gathered = _sc_gather(idx, table, tail)
    return _tc_transpose(gathered)
